# Initial kernel scaffold; baseline (speedup 1.0000x reference)
import jax
import jax.numpy as jnp
from jax.experimental import pallas as pl


def kernel(F, edge_index, W, att_src, att_dst, bias):
    raise NotImplementedError("write your pallas kernel here")


# speedup vs baseline: 85.4366x; 85.4366x over previous
import functools

import jax
import jax.numpy as jnp
from jax import lax
from jax.experimental import pallas as pl
from jax.experimental.pallas import tpu as pltpu
from jax.experimental.pallas import tpu_sc as plsc

NEG_INIT = -3.0e38
LANES = 16


def _take(v, idx):
    return lax.gather(
        v, idx[:, None],
        dimension_numbers=lax.GatherDimensionNumbers(
            offset_dims=(), collapsed_slice_dims=(0,),
            start_index_map=(0,)),
        slice_sizes=(1,),
        mode=lax.GatherScatterMode.PROMISE_IN_BOUNDS)


def _splat(v, j):
    return _take(v, jnp.zeros((LANES,), jnp.int32) + j)


def _matvec_body(f_ref, w_ref, o_ref):
    o_ref[:, :] = jnp.dot(f_ref[:, :], w_ref[:, :],
                          preferred_element_type=jnp.float32)


def _final_body(den_ref, num_ref, b_ref, o_ref):
    den = den_ref[0] + den_ref[1]
    num = num_ref[0] + num_ref[1]
    o_ref[:, :] = num / (den + 1e-16) + b_ref[0, 0]


def _seg_scan(keys, vals, iota, op):
    for sh in (1, 2, 4, 8):
        idx = jnp.maximum(iota - sh, 0)
        kp = _take(keys, idx)
        vp = _take(vals, idx)
        same = (kp == keys) & (iota >= sh)
        vals = jnp.where(same, op(vals, vp), vals)
    kn = _take(keys, jnp.minimum(iota + 1, LANES - 1))
    islast = (iota == LANES - 1) | (keys != kn)
    return vals, islast


def _make_sc_kernels(n_pad, e_per, nw, nt):
    nslice = n_pad // nt
    mesh = plsc.VectorSubcoreMesh(core_axis_name="c", subcore_axis_name="s")

    def _load_edges(x_hbm, src_hbm, dst_hbm, consts_hbm,
                    x_v, src_v, dst_v, consts_v):
        c = lax.axis_index("c")
        s = lax.axis_index("s")
        wid = c * nt + s
        ebase = wid * e_per
        pltpu.sync_copy(x_hbm, x_v)
        pltpu.sync_copy(src_hbm.at[pl.ds(ebase, e_per)], src_v)
        pltpu.sync_copy(dst_hbm.at[pl.ds(ebase, e_per)], dst_v)
        pltpu.sync_copy(consts_hbm, consts_v)
        cv = consts_v[...]
        return c, s, _splat(cv, 0), _splat(cv, 1)

    def _fill(ref, value):
        def body(i, _):
            ref[pl.ds(i * LANES, LANES)] = jnp.full((LANES,), value,
                                                    jnp.float32)
            return 0
        lax.fori_loop(0, n_pad // LANES, body, 0)

    def _combine(core, sid, priv_v, tmp_v, acc_v, sh_ref, out_hbm, op):
        nbase = sid * nslice
        pltpu.sync_copy(sh_ref.at[0, pl.ds(nbase, nslice)], acc_v)
        for t in range(1, nt):
            pltpu.sync_copy(sh_ref.at[t, pl.ds(nbase, nslice)], tmp_v)

            def body(j, _):
                a = acc_v[pl.ds(j * LANES, LANES)]
                b = tmp_v[pl.ds(j * LANES, LANES)]
                acc_v[pl.ds(j * LANES, LANES)] = op(a, b)
                return 0
            lax.fori_loop(0, nslice // LANES, body, 0)
        pltpu.sync_copy(acc_v, out_hbm.at[core, pl.ds(nbase, nslice)])

    @functools.partial(
        pl.kernel, mesh=mesh,
        compiler_params=pltpu.CompilerParams(needs_layout_passes=False),
        out_type=jax.ShapeDtypeStruct((2, n_pad), jnp.float32),
        scratch_types=[
            pltpu.VMEM((n_pad,), jnp.float32),
            pltpu.VMEM((e_per,), jnp.int32),
            pltpu.VMEM((e_per,), jnp.int32),
            pltpu.VMEM((n_pad,), jnp.float32),
            pltpu.VMEM((nslice,), jnp.float32),
            pltpu.VMEM((nslice,), jnp.float32),
            pltpu.VMEM((LANES,), jnp.float32),
            pltpu.VMEM_SHARED((nt, n_pad), jnp.float32),
        ],
    )
    def amax_kernel(x_hbm, src_hbm, dst_hbm, consts_hbm, amax_part_hbm,
                    x_v, src_v, dst_v, amax_v, tmp_v, acc_v, consts_v,
                    sh_ref):
        c, s, a_s, a_d = _load_edges(x_hbm, src_hbm, dst_hbm, consts_hbm,
                                     x_v, src_v, dst_v, consts_v)
        _fill(amax_v, NEG_INIT)
        iota = lax.iota(jnp.int32, LANES)

        def body(i, _):
            off = i * LANES
            sidx = src_v[pl.ds(off, LANES)]
            didx = dst_v[pl.ds(off, LANES)]
            xs = plsc.load_gather(x_v, [sidx])
            xd = plsc.load_gather(x_v, [didx])
            l = a_s * xs + a_d * xd
            alpha = jnp.maximum(l, 0.2 * l)
            ks, vs = plsc.sort_key_val(didx, alpha)
            vs, islast = _seg_scan(ks, vs, iota, jnp.maximum)
            cur = plsc.load_gather(amax_v, [ks])
            plsc.store_scatter(amax_v, [ks], jnp.maximum(cur, vs),
                               mask=islast)
            return 0
        lax.fori_loop(0, e_per // LANES, body, 0)

        pltpu.sync_copy(amax_v, sh_ref.at[s])
        plsc.subcore_barrier()
        _combine(c, s, amax_v, tmp_v, acc_v, sh_ref, amax_part_hbm,
                 jnp.maximum)

    @functools.partial(
        pl.kernel, mesh=mesh,
        compiler_params=pltpu.CompilerParams(needs_layout_passes=False),
        out_type=[jax.ShapeDtypeStruct((2, n_pad), jnp.float32),
                  jax.ShapeDtypeStruct((2, n_pad), jnp.float32)],
        scratch_types=[
            pltpu.VMEM((n_pad,), jnp.float32),
            pltpu.VMEM((e_per,), jnp.int32),
            pltpu.VMEM((e_per,), jnp.int32),
            pltpu.VMEM((n_pad,), jnp.float32),
            pltpu.VMEM((n_pad,), jnp.float32),
            pltpu.VMEM((n_pad,), jnp.float32),
            pltpu.VMEM((nslice,), jnp.float32),
            pltpu.VMEM((nslice,), jnp.float32),
            pltpu.VMEM((LANES,), jnp.float32),
            pltpu.VMEM_SHARED((nt, n_pad), jnp.float32),
            pltpu.VMEM_SHARED((nt, n_pad), jnp.float32),
        ],
    )
    def sum_kernel(x_hbm, src_hbm, dst_hbm, consts_hbm, amax_part_hbm,
                   den_part_hbm, num_part_hbm,
                   x_v, src_v, dst_v, amax_v, den_v, num_v, tmp_v, acc_v,
                   consts_v, shd_ref, shn_ref):
        c, s, a_s, a_d = _load_edges(x_hbm, src_hbm, dst_hbm, consts_hbm,
                                     x_v, src_v, dst_v, consts_v)
        pltpu.sync_copy(amax_part_hbm.at[0], amax_v)
        pltpu.sync_copy(amax_part_hbm.at[1], den_v)

        def maxbody(j, _):
            a = amax_v[pl.ds(j * LANES, LANES)]
            b = den_v[pl.ds(j * LANES, LANES)]
            amax_v[pl.ds(j * LANES, LANES)] = jnp.maximum(a, b)
            return 0
        lax.fori_loop(0, n_pad // LANES, maxbody, 0)
        _fill(den_v, 0.0)
        _fill(num_v, 0.0)
        iota = lax.iota(jnp.int32, LANES)

        def body(i, _):
            off = i * LANES
            sidx = src_v[pl.ds(off, LANES)]
            didx = dst_v[pl.ds(off, LANES)]
            xs = plsc.load_gather(x_v, [sidx])
            xd = plsc.load_gather(x_v, [didx])
            l = a_s * xs + a_d * xd
            alpha = jnp.maximum(l, 0.2 * l)
            am = plsc.load_gather(amax_v, [didx])
            e = jnp.exp(alpha - am)
            ks, perm = plsc.sort_key_val(didx, iota)
            es = _take(e, perm)
            xxs = _take(e * xs, perm)
            es, islast = _seg_scan(ks, es, iota, jnp.add)
            xxs, _ = _seg_scan(ks, xxs, iota, jnp.add)
            curd = plsc.load_gather(den_v, [ks])
            plsc.store_scatter(den_v, [ks], curd + es, mask=islast)
            curn = plsc.load_gather(num_v, [ks])
            plsc.store_scatter(num_v, [ks], curn + xxs, mask=islast)
            return 0
        lax.fori_loop(0, e_per // LANES, body, 0)

        pltpu.sync_copy(den_v, shd_ref.at[s])
        pltpu.sync_copy(num_v, shn_ref.at[s])
        plsc.subcore_barrier()
        _combine(c, s, den_v, tmp_v, acc_v, shd_ref, den_part_hbm, jnp.add)
        _combine(c, s, num_v, tmp_v, acc_v, shn_ref, num_part_hbm, jnp.add)

    return amax_kernel, sum_kernel


def kernel(F, edge_index, W, att_src, att_dst, bias):
    n, d = F.shape
    e = edge_index.shape[1]
    nw, nt = 32, 16
    n_pad = ((n + 16 * nt - 1) // (16 * nt)) * (16 * nt)
    e_per = e // nw

    f_pad = jnp.pad(F, ((0, n_pad - n), (0, 0)))
    x2 = pl.pallas_call(
        _matvec_body,
        out_shape=jax.ShapeDtypeStruct((n_pad, 1), jnp.float32),
    )(f_pad, W)
    x_flat = x2.reshape(n_pad)

    src = edge_index[0]
    dst = edge_index[1]
    consts = jnp.zeros((LANES,), jnp.float32)
    consts = consts.at[0].set(att_src[0, 0]).at[1].set(att_dst[0, 0])

    amax_kernel, sum_kernel = _make_sc_kernels(n_pad, e_per, nw, nt)
    amax_part = amax_kernel(x_flat, src, dst, consts)
    den_part, num_part = sum_kernel(x_flat, src, dst, consts, amax_part)

    out2 = pl.pallas_call(
        _final_body,
        out_shape=jax.ShapeDtypeStruct((n_pad // 128, 128), jnp.float32),
    )(den_part.reshape(2, n_pad // 128, 128),
      num_part.reshape(2, n_pad // 128, 128),
      bias.reshape(1, 1))
    return out2.reshape(n_pad)[:n]


# speedup vs baseline: 136.0235x; 1.5921x over previous
import functools

import jax
import jax.numpy as jnp
from jax import lax
from jax.experimental import pallas as pl
from jax.experimental.pallas import tpu as pltpu
from jax.experimental.pallas import tpu_sc as plsc

NEG_INIT = -3.0e38
LANES = 16
UNROLL = 5


def _take(v, idx):
    return lax.gather(
        v, idx[:, None],
        dimension_numbers=lax.GatherDimensionNumbers(
            offset_dims=(), collapsed_slice_dims=(0,),
            start_index_map=(0,)),
        slice_sizes=(1,),
        mode=lax.GatherScatterMode.PROMISE_IN_BOUNDS)


def _splat(v, j):
    return _take(v, jnp.zeros((LANES,), jnp.int32) + j)


def _matvec_body(f_ref, w_ref, o_ref):
    o_ref[:, :] = jnp.dot(f_ref[:, :], w_ref[:, :],
                          preferred_element_type=jnp.float32)


def _final_body(m_ref, den_ref, num_ref, b_ref, o_ref):
    m0 = m_ref[0]
    m1 = m_ref[1]
    mm = jnp.maximum(m0, m1)
    s0 = jnp.exp(m0 - mm)
    s1 = jnp.exp(m1 - mm)
    den = den_ref[0] * s0 + den_ref[1] * s1
    num = num_ref[0] * s0 + num_ref[1] * s1
    o_ref[:, :] = num / (den + 1e-16) + b_ref[0, 0]


def _seg_max_scan(keys, vals, iota):
    for sh in (1, 2, 4, 8):
        idx = jnp.maximum(iota - sh, 0)
        same = (_take(keys, idx) == keys) & (iota >= sh)
        vals = jnp.where(same, jnp.maximum(vals, _take(vals, idx)), vals)
    kn = _take(keys, jnp.minimum(iota + 1, LANES - 1))
    islast = (iota == LANES - 1) | (keys != kn)
    return vals, islast


def _seg_sum_scan2(keys, v1, v2, iota):
    for sh in (1, 2, 4, 8):
        idx = jnp.maximum(iota - sh, 0)
        same = (_take(keys, idx) == keys) & (iota >= sh)
        v1 = jnp.where(same, v1 + _take(v1, idx), v1)
        v2 = jnp.where(same, v2 + _take(v2, idx), v2)
    kn = _take(keys, jnp.minimum(iota + 1, LANES - 1))
    islast = (iota == LANES - 1) | (keys != kn)
    return v1, v2, islast


def _make_sc_kernel(n_pad, e_per, nt):
    nslice = n_pad // nt
    nvec = nslice // LANES
    mesh = plsc.VectorSubcoreMesh(core_axis_name="c", subcore_axis_name="s")
    part = jax.ShapeDtypeStruct((2, n_pad), jnp.float32)

    @functools.partial(
        pl.kernel, mesh=mesh,
        compiler_params=pltpu.CompilerParams(needs_layout_passes=False),
        out_type=[part, part, part],
        scratch_types=[
            pltpu.VMEM((n_pad,), jnp.float32),
            pltpu.VMEM((e_per,), jnp.int32),
            pltpu.VMEM((e_per,), jnp.int32),
            pltpu.VMEM((n_pad,), jnp.float32),
            pltpu.VMEM((n_pad,), jnp.float32),
            pltpu.VMEM((n_pad,), jnp.float32),
            pltpu.VMEM((nt * 3, nslice), jnp.float32),
            pltpu.VMEM((nslice,), jnp.float32),
            pltpu.VMEM((nslice,), jnp.float32),
            pltpu.VMEM((nslice,), jnp.float32),
            pltpu.VMEM((LANES,), jnp.float32),
            pltpu.VMEM_SHARED((nt, 3 * n_pad), jnp.float32),
            pltpu.SemaphoreType.DMA,
        ],
    )
    def sc_kernel(x_hbm, src_hbm, dst_hbm, consts_hbm,
                  m_part_hbm, den_part_hbm, num_part_hbm,
                  x_v, src_v, dst_v, amax_v, den_v, num_v,
                  gath_v, macc_v, dacc_v, nacc_v, consts_v, sh_ref, sem):
        c = lax.axis_index("c")
        s = lax.axis_index("s")
        wid = c * nt + s
        ebase = wid * e_per
        pltpu.sync_copy(x_hbm, x_v)
        pltpu.sync_copy(src_hbm.at[pl.ds(ebase, e_per)], src_v)
        pltpu.sync_copy(dst_hbm.at[pl.ds(ebase, e_per)], dst_v)
        pltpu.sync_copy(consts_hbm, consts_v)
        cv = consts_v[...]
        a_s = _splat(cv, 0)
        a_d = _splat(cv, 1)
        iota = lax.iota(jnp.int32, LANES)

        def fill2(ref1, ref2, value):
            def body(i, _):
                v = jnp.full((LANES,), value, jnp.float32)
                ref1[pl.ds(i * LANES, LANES)] = v
                ref2[pl.ds(i * LANES, LANES)] = v
                return 0
            lax.fori_loop(0, n_pad // LANES, body, 0)

        def fill1(ref, value):
            def body(i, _):
                ref[pl.ds(i * LANES, LANES)] = jnp.full((LANES,), value,
                                                        jnp.float32)
                return 0
            lax.fori_loop(0, n_pad // LANES, body, 0)

        fill1(amax_v, NEG_INIT)

        def pass1(i, _):
            base = i * (LANES * UNROLL)
            offs = [base + u * LANES for u in range(UNROLL)]
            sid = [src_v[pl.ds(o, LANES)] for o in offs]
            did = [dst_v[pl.ds(o, LANES)] for o in offs]
            updates = []
            for u in range(UNROLL):
                ks, ss = plsc.sort_key_val(did[u], sid[u])
                xs = plsc.load_gather(x_v, [ss])
                xd = plsc.load_gather(x_v, [ks])
                l = a_s * xs + a_d * xd
                alpha = jnp.maximum(l, 0.2 * l)
                vs, islast = _seg_max_scan(ks, alpha, iota)
                updates.append((ks, ss, alpha, vs, islast))
            for o, (ks, ss, alpha, vs, islast) in zip(offs, updates):
                dst_v[pl.ds(o, LANES)] = ks
                src_v[pl.ds(o, LANES)] = ss
            for ks, ss, alpha, vs, islast in updates:
                cur = plsc.load_gather(amax_v, [ks])
                plsc.store_scatter(amax_v, [ks], jnp.maximum(cur, vs),
                                   mask=islast)
            return 0
        lax.fori_loop(0, e_per // (LANES * UNROLL), pass1, 0)

        fill2(den_v, num_v, 0.0)

        def pass2(i, _):
            base = i * (LANES * UNROLL)
            offs = [base + u * LANES for u in range(UNROLL)]
            updates = []
            for o in offs:
                ks = dst_v[pl.ds(o, LANES)]
                ss = src_v[pl.ds(o, LANES)]
                xs = plsc.load_gather(x_v, [ss])
                xd = plsc.load_gather(x_v, [ks])
                l = a_s * xs + a_d * xd
                alpha = jnp.maximum(l, 0.2 * l)
                am = plsc.load_gather(amax_v, [ks])
                e = jnp.exp(alpha - am)
                es, xxs, islast = _seg_sum_scan2(ks, e, e * xs, iota)
                updates.append((ks, es, xxs, islast))
            for ks, es, xxs, islast in updates:
                curd = plsc.load_gather(den_v, [ks])
                plsc.store_scatter(den_v, [ks], curd + es, mask=islast)
            for ks, es, xxs, islast in updates:
                curn = plsc.load_gather(num_v, [ks])
                plsc.store_scatter(num_v, [ks], curn + xxs, mask=islast)
            return 0
        lax.fori_loop(0, e_per // (LANES * UNROLL), pass2, 0)

        pltpu.sync_copy(amax_v, sh_ref.at[s, pl.ds(0, n_pad)])
        pltpu.sync_copy(den_v, sh_ref.at[s, pl.ds(n_pad, n_pad)])
        pltpu.sync_copy(num_v, sh_ref.at[s, pl.ds(2 * n_pad, n_pad)])
        plsc.subcore_barrier()

        nbase = s * nslice
        copies = []
        for t in range(nt):
            for k in range(3):
                copies.append(pltpu.async_copy(
                    sh_ref.at[t, pl.ds(k * n_pad + nbase, nslice)],
                    gath_v.at[t * 3 + k], sem))
        for cp in copies:
            cp.wait()

        def comb(j, _):
            sl = pl.ds(j * LANES, LANES)
            mm = gath_v[0, sl]
            for t in range(1, nt):
                mm = jnp.maximum(mm, gath_v[t * 3, sl])
            dacc = jnp.zeros((LANES,), jnp.float32)
            nacc = jnp.zeros((LANES,), jnp.float32)
            for t in range(nt):
                sc = jnp.exp(gath_v[t * 3, sl] - mm)
                dacc = dacc + gath_v[t * 3 + 1, sl] * sc
                nacc = nacc + gath_v[t * 3 + 2, sl] * sc
            macc_v[sl] = mm
            dacc_v[sl] = dacc
            nacc_v[sl] = nacc
            return 0
        lax.fori_loop(0, nvec, comb, 0)

        pltpu.sync_copy(macc_v, m_part_hbm.at[c, pl.ds(nbase, nslice)])
        pltpu.sync_copy(dacc_v, den_part_hbm.at[c, pl.ds(nbase, nslice)])
        pltpu.sync_copy(nacc_v, num_part_hbm.at[c, pl.ds(nbase, nslice)])

    return sc_kernel


def kernel(F, edge_index, W, att_src, att_dst, bias):
    n, d = F.shape
    e = edge_index.shape[1]
    nw, nt = 32, 16
    n_pad = ((n + 16 * nt - 1) // (16 * nt)) * (16 * nt)
    e_per = e // nw

    f_pad = jnp.pad(F, ((0, n_pad - n), (0, 0)))
    x2 = pl.pallas_call(
        _matvec_body,
        out_shape=jax.ShapeDtypeStruct((n_pad, 1), jnp.float32),
    )(f_pad, W)
    x_flat = x2.reshape(n_pad)

    src = edge_index[0]
    dst = edge_index[1]
    consts = jnp.zeros((LANES,), jnp.float32)
    consts = consts.at[0].set(att_src[0, 0]).at[1].set(att_dst[0, 0])

    sc_kernel = _make_sc_kernel(n_pad, e_per, nt)
    m_part, den_part, num_part = sc_kernel(x_flat, src, dst, consts)

    out2 = pl.pallas_call(
        _final_body,
        out_shape=jax.ShapeDtypeStruct((n_pad // 128, 128), jnp.float32),
    )(m_part.reshape(2, n_pad // 128, 128),
      den_part.reshape(2, n_pad // 128, 128),
      num_part.reshape(2, n_pad // 128, 128),
      bias.reshape(1, 1))
    return out2.reshape(n_pad)[:n]


# speedup vs baseline: 177.4903x; 1.3048x over previous
import functools

import jax
import jax.numpy as jnp
from jax import lax
from jax.experimental import pallas as pl
from jax.experimental.pallas import tpu as pltpu
from jax.experimental.pallas import tpu_sc as plsc

NEG_INIT = -3.0e38
LANES = 16
UNROLL = 5


def _take(v, idx):
    return lax.gather(
        v, idx[:, None],
        dimension_numbers=lax.GatherDimensionNumbers(
            offset_dims=(), collapsed_slice_dims=(0,),
            start_index_map=(0,)),
        slice_sizes=(1,),
        mode=lax.GatherScatterMode.PROMISE_IN_BOUNDS)


def _splat(v, j):
    return _take(v, jnp.zeros((LANES,), jnp.int32) + j)


def _matvec_body(f_ref, w_ref, as_ref, ad_ref, o_ref):
    n = f_ref.shape[0]
    o_ref[pl.ds(0, n), :] = jnp.dot(f_ref[:, :], w_ref[:, :],
                                    preferred_element_type=jnp.float32)
    o_ref[pl.ds(n, 8), :] = jnp.full((8, 1), 1.0) * as_ref[0, 0]
    o_ref[pl.ds(n + 8, 8), :] = jnp.full((8, 1), 1.0) * ad_ref[0, 0]


def _final_body(m_ref, den_ref, num_ref, b_ref, o_ref):
    m0 = m_ref[0]
    m1 = m_ref[1]
    mm = jnp.maximum(m0, m1)
    s0 = jnp.exp(m0 - mm)
    s1 = jnp.exp(m1 - mm)
    den = den_ref[0] * s0 + den_ref[1] * s1
    num = num_ref[0] * s0 + num_ref[1] * s1
    o_ref[:, :] = num / (den + 1e-16) + b_ref[0, 0]


def _islast(keys, iota):
    kn = _take(keys, jnp.minimum(iota + 1, LANES - 1))
    return (iota == LANES - 1) | (keys != kn)


def _seg_max_scan(keys, vals, iota):
    for sh in (1, 2, 4, 8):
        idx = jnp.maximum(iota - sh, 0)
        same = (_take(keys, idx) == keys) & (iota >= sh)
        vals = jnp.where(same, jnp.maximum(vals, _take(vals, idx)), vals)
    return vals


def _seg_sum_scan2(keys, v1, v2, iota):
    for sh in (1, 2, 4, 8):
        idx = jnp.maximum(iota - sh, 0)
        same = (_take(keys, idx) == keys) & (iota >= sh)
        v1 = jnp.where(same, v1 + _take(v1, idx), v1)
        v2 = jnp.where(same, v2 + _take(v2, idx), v2)
    return v1, v2


def _make_sc_kernel(n_pad, e_per, nt, n_edges, n_real):
    nslice = n_pad // nt
    nvec = nslice // LANES
    mesh = plsc.VectorSubcoreMesh(core_axis_name="c", subcore_axis_name="s")
    part = jax.ShapeDtypeStruct((2, n_pad), jnp.float32)

    @functools.partial(
        pl.kernel, mesh=mesh,
        compiler_params=pltpu.CompilerParams(needs_layout_passes=False),
        out_type=[part, part, part],
        scratch_types=[
            pltpu.VMEM((n_pad,), jnp.float32),
            pltpu.VMEM((e_per,), jnp.int32),
            pltpu.VMEM((e_per,), jnp.int32),
            pltpu.VMEM((n_pad,), jnp.float32),
            pltpu.VMEM((n_pad,), jnp.float32),
            pltpu.VMEM((n_pad,), jnp.float32),
            pltpu.VMEM((nt * 3, nslice), jnp.float32),
            pltpu.VMEM((nslice,), jnp.float32),
            pltpu.VMEM((nslice,), jnp.float32),
            pltpu.VMEM((nslice,), jnp.float32),
            pltpu.VMEM_SHARED((nt, 3 * n_pad), jnp.float32),
            pltpu.SemaphoreType.DMA,
        ],
    )
    def sc_kernel(x_hbm, edge_hbm,
                  m_part_hbm, den_part_hbm, num_part_hbm,
                  x_v, src_v, dst_v, amax_v, den_v, num_v,
                  gath_v, macc_v, dacc_v, nacc_v, sh_ref, sem):
        c = lax.axis_index("c")
        s = lax.axis_index("s")
        wid = c * nt + s
        ebase = wid * e_per
        pltpu.sync_copy(x_hbm, x_v)
        pltpu.sync_copy(edge_hbm.at[pl.ds(ebase, e_per)], src_v)
        pltpu.sync_copy(edge_hbm.at[pl.ds(n_edges + ebase, e_per)], dst_v)
        zero16 = jnp.zeros((LANES,), jnp.int32)
        a_s = plsc.load_gather(x_v, [zero16 + n_real])
        a_d = plsc.load_gather(x_v, [zero16 + (n_real + 8)])
        iota = lax.iota(jnp.int32, LANES)

        def fill(refs, value):
            fu = 8
            def body(i, _):
                v = jnp.full((LANES,), value, jnp.float32)
                for ref in refs:
                    for u in range(fu):
                        ref[pl.ds((i * fu + u) * LANES, LANES)] = v
                return 0
            lax.fori_loop(0, n_pad // (LANES * fu), body, 0)

        fill([amax_v], NEG_INIT)

        def pass1(i, _):
            base = i * (LANES * UNROLL)
            offs = [base + u * LANES for u in range(UNROLL)]
            sid = [src_v[pl.ds(o, LANES)] for o in offs]
            did = [dst_v[pl.ds(o, LANES)] for o in offs]
            kss = []
            alphas = []
            masks = []
            for u in range(UNROLL):
                ks, ss = plsc.sort_key_val(did[u], sid[u])
                xs = plsc.load_gather(x_v, [ss])
                xd = plsc.load_gather(x_v, [ks])
                l = a_s * xs + a_d * xd
                alpha = jnp.maximum(l, 0.2 * l)
                kss.append((ks, ss))
                alphas.append(alpha)
                masks.append(_islast(ks, iota))
            for o, (ks, ss) in zip(offs, kss):
                dst_v[pl.ds(o, LANES)] = ks
                src_v[pl.ds(o, LANES)] = ss
            nodup = functools.reduce(
                lambda a, b: a & b, [jnp.all(m) for m in masks])
            vss = lax.cond(
                nodup,
                lambda: tuple(alphas),
                lambda: tuple(_seg_max_scan(kss[u][0], alphas[u], iota)
                              for u in range(UNROLL)))
            for (ks, ss), vs, islast in zip(kss, vss, masks):
                cur = plsc.load_gather(amax_v, [ks])
                plsc.store_scatter(amax_v, [ks], jnp.maximum(cur, vs),
                                   mask=islast)
            return 0
        lax.fori_loop(0, e_per // (LANES * UNROLL), pass1, 0)

        fill([den_v, num_v], 0.0)

        def pass2(i, _):
            base = i * (LANES * UNROLL)
            offs = [base + u * LANES for u in range(UNROLL)]
            kslist = []
            evals = []
            masks = []
            for o in offs:
                ks = dst_v[pl.ds(o, LANES)]
                ss = src_v[pl.ds(o, LANES)]
                xs = plsc.load_gather(x_v, [ss])
                xd = plsc.load_gather(x_v, [ks])
                l = a_s * xs + a_d * xd
                alpha = jnp.maximum(l, 0.2 * l)
                am = plsc.load_gather(amax_v, [ks])
                e = jnp.exp(alpha - am)
                kslist.append(ks)
                evals.append((e, e * xs))
                masks.append(_islast(ks, iota))
            nodup = functools.reduce(
                lambda a, b: a & b, [jnp.all(m) for m in masks])

            def scanned():
                out = []
                for u in range(UNROLL):
                    es, xxs = _seg_sum_scan2(kslist[u], evals[u][0],
                                             evals[u][1], iota)
                    out.extend((es, xxs))
                return tuple(out)

            flat = lax.cond(
                nodup,
                lambda: tuple(v for ev in evals for v in ev),
                scanned)
            for u in range(UNROLL):
                plsc.addupdate_scatter(den_v, [kslist[u]], flat[2 * u],
                                       mask=masks[u])
                plsc.addupdate_scatter(num_v, [kslist[u]],
                                       flat[2 * u + 1], mask=masks[u])
            return 0
        lax.fori_loop(0, e_per // (LANES * UNROLL), pass2, 0)

        pltpu.sync_copy(amax_v, sh_ref.at[s, pl.ds(0, n_pad)])
        pltpu.sync_copy(den_v, sh_ref.at[s, pl.ds(n_pad, n_pad)])
        pltpu.sync_copy(num_v, sh_ref.at[s, pl.ds(2 * n_pad, n_pad)])
        plsc.subcore_barrier()

        nbase = s * nslice
        copies = []
        for t in range(nt):
            for k in range(3):
                copies.append(pltpu.async_copy(
                    sh_ref.at[t, pl.ds(k * n_pad + nbase, nslice)],
                    gath_v.at[t * 3 + k], sem))
        for cp in copies:
            cp.wait()

        def comb(j, _):
            sl = pl.ds(j * LANES, LANES)
            mm = gath_v[0, sl]
            for t in range(1, nt):
                mm = jnp.maximum(mm, gath_v[t * 3, sl])
            dacc = jnp.zeros((LANES,), jnp.float32)
            nacc = jnp.zeros((LANES,), jnp.float32)
            for t in range(nt):
                sc = jnp.exp(gath_v[t * 3, sl] - mm)
                dacc = dacc + gath_v[t * 3 + 1, sl] * sc
                nacc = nacc + gath_v[t * 3 + 2, sl] * sc
            macc_v[sl] = mm
            dacc_v[sl] = dacc
            nacc_v[sl] = nacc
            return 0
        lax.fori_loop(0, nvec, comb, 0)

        pltpu.sync_copy(macc_v, m_part_hbm.at[c, pl.ds(nbase, nslice)])
        pltpu.sync_copy(dacc_v, den_part_hbm.at[c, pl.ds(nbase, nslice)])
        pltpu.sync_copy(nacc_v, num_part_hbm.at[c, pl.ds(nbase, nslice)])

    return sc_kernel


def kernel(F, edge_index, W, att_src, att_dst, bias):
    n, d = F.shape
    e = edge_index.shape[1]
    nw, nt = 32, 16
    n_pad = ((n + 16 * nt - 1) // (16 * nt)) * (16 * nt)
    e_per = e // nw

    x2 = pl.pallas_call(
        _matvec_body,
        out_shape=jax.ShapeDtypeStruct((n_pad, 1), jnp.float32),
    )(F, W, att_src.reshape(1, 1), att_dst.reshape(1, 1))
    x_flat = x2.reshape(n_pad)

    sc_kernel = _make_sc_kernel(n_pad, e_per, nt, e, n)
    m_part, den_part, num_part = sc_kernel(x_flat, edge_index.reshape(2 * e))

    out2 = pl.pallas_call(
        _final_body,
        out_shape=jax.ShapeDtypeStruct((n_pad // 128, 128), jnp.float32),
    )(m_part.reshape(2, n_pad // 128, 128),
      den_part.reshape(2, n_pad // 128, 128),
      num_part.reshape(2, n_pad // 128, 128),
      bias.reshape(1, 1))
    return out2.reshape(n_pad)[:n]


# speedup vs baseline: 183.6944x; 1.0350x over previous
import functools

import jax
import jax.numpy as jnp
from jax import lax
from jax.experimental import pallas as pl
from jax.experimental.pallas import tpu as pltpu
from jax.experimental.pallas import tpu_sc as plsc

NEG_INIT = -3.0e38
LANES = 16
UNROLL = 5


def _take(v, idx):
    return lax.gather(
        v, idx[:, None],
        dimension_numbers=lax.GatherDimensionNumbers(
            offset_dims=(), collapsed_slice_dims=(0,),
            start_index_map=(0,)),
        slice_sizes=(1,),
        mode=lax.GatherScatterMode.PROMISE_IN_BOUNDS)


def _splat(v, j):
    return _take(v, jnp.zeros((LANES,), jnp.int32) + j)


def _matvec_body(f_ref, w_ref, as_ref, ad_ref, o_ref):
    n = f_ref.shape[0]
    o_ref[pl.ds(0, n), :] = jnp.dot(f_ref[:, :], w_ref[:, :],
                                    preferred_element_type=jnp.float32)
    o_ref[pl.ds(n, 8), :] = jnp.full((8, 1), 1.0) * as_ref[0, 0]
    o_ref[pl.ds(n + 8, 8), :] = jnp.full((8, 1), 1.0) * ad_ref[0, 0]


def _final_body(m_ref, den_ref, num_ref, b_ref, o_ref):
    m0 = m_ref[0]
    m1 = m_ref[1]
    mm = jnp.maximum(m0, m1)
    s0 = jnp.exp(m0 - mm)
    s1 = jnp.exp(m1 - mm)
    den = den_ref[0] * s0 + den_ref[1] * s1
    num = num_ref[0] * s0 + num_ref[1] * s1
    o_ref[:, :] = num / (den + 1e-16) + b_ref[0, 0]


def _islast(keys, iota):
    kn = _take(keys, jnp.minimum(iota + 1, LANES - 1))
    return (iota == LANES - 1) | (keys != kn)


def _seg_max_scan(keys, vals, iota):
    for sh in (1, 2, 4, 8):
        idx = jnp.maximum(iota - sh, 0)
        same = (_take(keys, idx) == keys) & (iota >= sh)
        vals = jnp.where(same, jnp.maximum(vals, _take(vals, idx)), vals)
    return vals


def _seg_sum_scan2(keys, v1, v2, iota):
    for sh in (1, 2, 4, 8):
        idx = jnp.maximum(iota - sh, 0)
        same = (_take(keys, idx) == keys) & (iota >= sh)
        v1 = jnp.where(same, v1 + _take(v1, idx), v1)
        v2 = jnp.where(same, v2 + _take(v2, idx), v2)
    return v1, v2


def _make_sc_kernel(n_pad, e_per, nt, n_edges, n_real):
    nslice = n_pad // nt
    nvec = nslice // LANES
    mesh = plsc.VectorSubcoreMesh(core_axis_name="c", subcore_axis_name="s")
    part = jax.ShapeDtypeStruct((2, n_pad), jnp.float32)

    @functools.partial(
        pl.kernel, mesh=mesh,
        compiler_params=pltpu.CompilerParams(needs_layout_passes=False),
        out_type=[part, part, part],
        scratch_types=[
            pltpu.VMEM((n_pad,), jnp.float32),
            pltpu.VMEM((e_per,), jnp.int32),
            pltpu.VMEM((e_per,), jnp.int32),
            pltpu.VMEM((n_pad,), jnp.float32),
            pltpu.VMEM((n_pad,), jnp.float32),
            pltpu.VMEM((n_pad,), jnp.float32),
            pltpu.VMEM((nt * 3, nslice), jnp.float32),
            pltpu.VMEM((nslice,), jnp.float32),
            pltpu.VMEM((nslice,), jnp.float32),
            pltpu.VMEM((nslice,), jnp.float32),
            pltpu.VMEM_SHARED((nt, 3 * n_pad), jnp.float32),
            pltpu.SemaphoreType.DMA,
        ],
    )
    def sc_kernel(x_hbm, edge_hbm,
                  m_part_hbm, den_part_hbm, num_part_hbm,
                  x_v, src_v, dst_v, amax_v, den_v, num_v,
                  gath_v, macc_v, dacc_v, nacc_v, sh_ref, sem):
        c = lax.axis_index("c")
        s = lax.axis_index("s")
        wid = c * nt + s
        ebase = wid * e_per
        pltpu.sync_copy(x_hbm, x_v)
        pltpu.sync_copy(edge_hbm.at[pl.ds(ebase, e_per)], src_v)
        pltpu.sync_copy(edge_hbm.at[pl.ds(n_edges + ebase, e_per)], dst_v)
        zero16 = jnp.zeros((LANES,), jnp.int32)
        a_s = plsc.load_gather(x_v, [zero16 + n_real])
        a_d = plsc.load_gather(x_v, [zero16 + (n_real + 8)])
        iota = lax.iota(jnp.int32, LANES)

        def fill(refs, value):
            fu = 8
            def body(i, _):
                v = jnp.full((LANES,), value, jnp.float32)
                for ref in refs:
                    for u in range(fu):
                        ref[pl.ds((i * fu + u) * LANES, LANES)] = v
                return 0
            lax.fori_loop(0, n_pad // (LANES * fu), body, 0)

        fill([amax_v], NEG_INIT)

        def pass1(i, _):
            base = i * (LANES * UNROLL)
            offs = [base + u * LANES for u in range(UNROLL)]
            sid = [src_v[pl.ds(o, LANES)] for o in offs]
            did = [dst_v[pl.ds(o, LANES)] for o in offs]
            kss = []
            alphas = []
            masks = []
            for u in range(UNROLL):
                ku, ss = plsc.sort_key_val(plsc.bitcast(did[u], jnp.uint32),
                                           sid[u])
                ks = plsc.bitcast(ku, jnp.int32)
                xs = plsc.load_gather(x_v, [ss])
                xd = plsc.load_gather(x_v, [ks])
                l = a_s * xs + a_d * xd
                alpha = jnp.maximum(l, 0.2 * l)
                kss.append((ks, ss))
                alphas.append(alpha)
                masks.append(_islast(ks, iota))
            for o, (ks, ss) in zip(offs, kss):
                dst_v[pl.ds(o, LANES)] = ks
                src_v[pl.ds(o, LANES)] = ss
            nodup = jnp.all(functools.reduce(lambda a, b: a & b, masks))
            vss = lax.cond(
                nodup,
                lambda: tuple(alphas),
                lambda: tuple(_seg_max_scan(kss[u][0], alphas[u], iota)
                              for u in range(UNROLL)))
            for (ks, ss), vs, islast in zip(kss, vss, masks):
                cur = plsc.load_gather(amax_v, [ks])
                plsc.store_scatter(amax_v, [ks], jnp.maximum(cur, vs),
                                   mask=islast)
            return 0
        lax.fori_loop(0, e_per // (LANES * UNROLL), pass1, 0)

        fill([den_v, num_v], 0.0)

        def pass2(i, _):
            base = i * (LANES * UNROLL)
            offs = [base + u * LANES for u in range(UNROLL)]
            kslist = []
            evals = []
            masks = []
            for o in offs:
                ks = dst_v[pl.ds(o, LANES)]
                ss = src_v[pl.ds(o, LANES)]
                xs = plsc.load_gather(x_v, [ss])
                xd = plsc.load_gather(x_v, [ks])
                l = a_s * xs + a_d * xd
                alpha = jnp.maximum(l, 0.2 * l)
                am = plsc.load_gather(amax_v, [ks])
                e = jnp.exp(alpha - am)
                kslist.append(ks)
                evals.append((e, e * xs))
                masks.append(_islast(ks, iota))
            nodup = jnp.all(functools.reduce(lambda a, b: a & b, masks))

            def scanned():
                out = []
                for u in range(UNROLL):
                    es, xxs = _seg_sum_scan2(kslist[u], evals[u][0],
                                             evals[u][1], iota)
                    out.extend((es, xxs))
                return tuple(out)

            flat = lax.cond(
                nodup,
                lambda: tuple(v for ev in evals for v in ev),
                scanned)
            for u in range(UNROLL):
                plsc.addupdate_scatter(den_v, [kslist[u]], flat[2 * u],
                                       mask=masks[u])
                plsc.addupdate_scatter(num_v, [kslist[u]],
                                       flat[2 * u + 1], mask=masks[u])
            return 0
        lax.fori_loop(0, e_per // (LANES * UNROLL), pass2, 0)

        pltpu.sync_copy(amax_v, sh_ref.at[s, pl.ds(0, n_pad)])
        pltpu.sync_copy(den_v, sh_ref.at[s, pl.ds(n_pad, n_pad)])
        pltpu.sync_copy(num_v, sh_ref.at[s, pl.ds(2 * n_pad, n_pad)])
        plsc.subcore_barrier()

        nbase = s * nslice
        copies = []
        for t in range(nt):
            for k in range(3):
                copies.append(pltpu.async_copy(
                    sh_ref.at[t, pl.ds(k * n_pad + nbase, nslice)],
                    gath_v.at[t * 3 + k], sem))
        for cp in copies:
            cp.wait()

        def comb(j, _):
            sl = pl.ds(j * LANES, LANES)
            mm = gath_v[0, sl]
            for t in range(1, nt):
                mm = jnp.maximum(mm, gath_v[t * 3, sl])
            dacc = jnp.zeros((LANES,), jnp.float32)
            nacc = jnp.zeros((LANES,), jnp.float32)
            for t in range(nt):
                sc = jnp.exp(gath_v[t * 3, sl] - mm)
                dacc = dacc + gath_v[t * 3 + 1, sl] * sc
                nacc = nacc + gath_v[t * 3 + 2, sl] * sc
            macc_v[sl] = mm
            dacc_v[sl] = dacc
            nacc_v[sl] = nacc
            return 0
        lax.fori_loop(0, nvec, comb, 0)

        pltpu.sync_copy(macc_v, m_part_hbm.at[c, pl.ds(nbase, nslice)])
        pltpu.sync_copy(dacc_v, den_part_hbm.at[c, pl.ds(nbase, nslice)])
        pltpu.sync_copy(nacc_v, num_part_hbm.at[c, pl.ds(nbase, nslice)])

    return sc_kernel


def kernel(F, edge_index, W, att_src, att_dst, bias):
    n, d = F.shape
    e = edge_index.shape[1]
    nw, nt = 32, 16
    n_pad = ((n + 16 * nt - 1) // (16 * nt)) * (16 * nt)
    e_per = e // nw

    x2 = pl.pallas_call(
        _matvec_body,
        out_shape=jax.ShapeDtypeStruct((n_pad, 1), jnp.float32),
    )(F, W, att_src.reshape(1, 1), att_dst.reshape(1, 1))
    x_flat = x2.reshape(n_pad)

    sc_kernel = _make_sc_kernel(n_pad, e_per, nt, e, n)
    m_part, den_part, num_part = sc_kernel(x_flat, edge_index.reshape(2 * e))

    out2 = pl.pallas_call(
        _final_body,
        out_shape=jax.ShapeDtypeStruct((n_pad // 128, 128), jnp.float32),
    )(m_part.reshape(2, n_pad // 128, 128),
      den_part.reshape(2, n_pad // 128, 128),
      num_part.reshape(2, n_pad // 128, 128),
      bias.reshape(1, 1))
    return out2.reshape(n_pad)[:n]


# speedup vs baseline: 187.8132x; 1.0224x over previous
import functools

import jax
import jax.numpy as jnp
from jax import lax
from jax.experimental import pallas as pl
from jax.experimental.pallas import tpu as pltpu
from jax.experimental.pallas import tpu_sc as plsc

NEG_INIT = -3.0e38
LANES = 16
UNROLL = 5


def _take(v, idx):
    return lax.gather(
        v, idx[:, None],
        dimension_numbers=lax.GatherDimensionNumbers(
            offset_dims=(), collapsed_slice_dims=(0,),
            start_index_map=(0,)),
        slice_sizes=(1,),
        mode=lax.GatherScatterMode.PROMISE_IN_BOUNDS)


def _splat(v, j):
    return _take(v, jnp.zeros((LANES,), jnp.int32) + j)


def _matvec_body(f_ref, w_ref, as_ref, ad_ref, o_ref):
    n = f_ref.shape[0]
    o_ref[pl.ds(0, n), :] = jnp.dot(f_ref[:, :], w_ref[:, :],
                                    preferred_element_type=jnp.float32)
    o_ref[pl.ds(n, 8), :] = jnp.full((8, 1), 1.0) * as_ref[0, 0]
    o_ref[pl.ds(n + 8, 8), :] = jnp.full((8, 1), 1.0) * ad_ref[0, 0]


def _final_body(m_ref, den_ref, num_ref, b_ref, o_ref):
    m0 = m_ref[0]
    m1 = m_ref[1]
    mm = jnp.maximum(m0, m1)
    s0 = jnp.exp(m0 - mm)
    s1 = jnp.exp(m1 - mm)
    den = den_ref[0] * s0 + den_ref[1] * s1
    num = num_ref[0] * s0 + num_ref[1] * s1
    o_ref[:, :] = num / (den + 1e-16) + b_ref[0, 0]


def _islast(keys, iota):
    kn = _take(keys, jnp.minimum(iota + 1, LANES - 1))
    return (iota == LANES - 1) | (keys != kn)


def _seg_max_scan(keys, vals, iota):
    for sh in (1, 2, 4, 8):
        idx = jnp.maximum(iota - sh, 0)
        same = (_take(keys, idx) == keys) & (iota >= sh)
        vals = jnp.where(same, jnp.maximum(vals, _take(vals, idx)), vals)
    return vals


def _seg_sum_scan2(keys, v1, v2, iota):
    for sh in (1, 2, 4, 8):
        idx = jnp.maximum(iota - sh, 0)
        same = (_take(keys, idx) == keys) & (iota >= sh)
        v1 = jnp.where(same, v1 + _take(v1, idx), v1)
        v2 = jnp.where(same, v2 + _take(v2, idx), v2)
    return v1, v2


def _make_sc_kernel(n_pad, e_per, nt, n_edges, n_real):
    nslice = n_pad // nt
    nvec = nslice // LANES
    mesh = plsc.VectorSubcoreMesh(core_axis_name="c", subcore_axis_name="s")
    part = jax.ShapeDtypeStruct((2, n_pad), jnp.float32)

    @functools.partial(
        pl.kernel, mesh=mesh,
        compiler_params=pltpu.CompilerParams(needs_layout_passes=False),
        out_type=[part, part, part],
        scratch_types=[
            pltpu.VMEM((n_pad,), jnp.float32),
            pltpu.VMEM((e_per,), jnp.int32),
            pltpu.VMEM((e_per,), jnp.int32),
            pltpu.VMEM((n_pad,), jnp.float32),
            pltpu.VMEM((n_pad,), jnp.float32),
            pltpu.VMEM((n_pad,), jnp.float32),
            pltpu.VMEM((nt * 3, nslice), jnp.float32),
            pltpu.VMEM((nslice,), jnp.float32),
            pltpu.VMEM((nslice,), jnp.float32),
            pltpu.VMEM((nslice,), jnp.float32),
            pltpu.VMEM_SHARED((nt, 3 * n_pad), jnp.float32),
            pltpu.SemaphoreType.DMA,
        ],
    )
    def sc_kernel(x_hbm, edge_hbm,
                  m_part_hbm, den_part_hbm, num_part_hbm,
                  x_v, src_v, dst_v, amax_v, den_v, num_v,
                  gath_v, macc_v, dacc_v, nacc_v, sh_ref, sem):
        c = lax.axis_index("c")
        s = lax.axis_index("s")
        wid = c * nt + s
        ebase = wid * e_per
        ins = [
            pltpu.async_copy(x_hbm, x_v, sem),
            pltpu.async_copy(edge_hbm.at[pl.ds(ebase, e_per)], src_v, sem),
            pltpu.async_copy(edge_hbm.at[pl.ds(n_edges + ebase, e_per)],
                             dst_v, sem),
        ]
        for cp in ins:
            cp.wait()
        zero16 = jnp.zeros((LANES,), jnp.int32)
        a_s = plsc.load_gather(x_v, [zero16 + n_real])
        a_d = plsc.load_gather(x_v, [zero16 + (n_real + 8)])
        iota = lax.iota(jnp.int32, LANES)

        def fill(refs, value):
            fu = 8
            def body(i, _):
                v = jnp.full((LANES,), value, jnp.float32)
                for ref in refs:
                    for u in range(fu):
                        ref[pl.ds((i * fu + u) * LANES, LANES)] = v
                return 0
            lax.fori_loop(0, n_pad // (LANES * fu), body, 0)

        fill([amax_v], NEG_INIT)

        def pass1(i, _):
            base = i * (LANES * UNROLL)
            offs = [base + u * LANES for u in range(UNROLL)]
            sid = [src_v[pl.ds(o, LANES)] for o in offs]
            did = [dst_v[pl.ds(o, LANES)] for o in offs]
            kss = []
            alphas = []
            masks = []
            for u in range(UNROLL):
                ku, ss = plsc.sort_key_val(plsc.bitcast(did[u], jnp.uint32),
                                           sid[u])
                ks = plsc.bitcast(ku, jnp.int32)
                xs = plsc.load_gather(x_v, [ss])
                xd = plsc.load_gather(x_v, [ks])
                l = a_s * xs + a_d * xd
                alpha = jnp.maximum(l, 0.2 * l)
                kss.append((ks, ss))
                alphas.append(alpha)
                masks.append(_islast(ks, iota))
            for o, (ks, ss) in zip(offs, kss):
                dst_v[pl.ds(o, LANES)] = ks
                src_v[pl.ds(o, LANES)] = ss
            nodup = jnp.all(functools.reduce(lambda a, b: a & b, masks))
            vss = lax.cond(
                nodup,
                lambda: tuple(alphas),
                lambda: tuple(_seg_max_scan(kss[u][0], alphas[u], iota)
                              for u in range(UNROLL)))
            for (ks, ss), vs, islast in zip(kss, vss, masks):
                cur = plsc.load_gather(amax_v, [ks])
                plsc.store_scatter(amax_v, [ks], jnp.maximum(cur, vs),
                                   mask=islast)
            return 0
        lax.fori_loop(0, e_per // (LANES * UNROLL), pass1, 0)

        fill([den_v, num_v], 0.0)

        def pass2(i, _):
            base = i * (LANES * UNROLL)
            offs = [base + u * LANES for u in range(UNROLL)]
            kslist = []
            evals = []
            masks = []
            for o in offs:
                ks = dst_v[pl.ds(o, LANES)]
                ss = src_v[pl.ds(o, LANES)]
                xs = plsc.load_gather(x_v, [ss])
                xd = plsc.load_gather(x_v, [ks])
                l = a_s * xs + a_d * xd
                alpha = jnp.maximum(l, 0.2 * l)
                am = plsc.load_gather(amax_v, [ks])
                e = jnp.exp(alpha - am)
                kslist.append(ks)
                evals.append((e, e * xs))
                masks.append(_islast(ks, iota))
            nodup = jnp.all(functools.reduce(lambda a, b: a & b, masks))

            def scanned():
                out = []
                for u in range(UNROLL):
                    es, xxs = _seg_sum_scan2(kslist[u], evals[u][0],
                                             evals[u][1], iota)
                    out.extend((es, xxs))
                return tuple(out)

            flat = lax.cond(
                nodup,
                lambda: tuple(v for ev in evals for v in ev),
                scanned)
            for u in range(UNROLL):
                plsc.addupdate_scatter(den_v, [kslist[u]], flat[2 * u],
                                       mask=masks[u])
                plsc.addupdate_scatter(num_v, [kslist[u]],
                                       flat[2 * u + 1], mask=masks[u])
            return 0
        lax.fori_loop(0, e_per // (LANES * UNROLL), pass2, 0)

        pubs = [
            pltpu.async_copy(amax_v, sh_ref.at[s, pl.ds(0, n_pad)], sem),
            pltpu.async_copy(den_v, sh_ref.at[s, pl.ds(n_pad, n_pad)], sem),
            pltpu.async_copy(num_v, sh_ref.at[s, pl.ds(2 * n_pad, n_pad)],
                             sem),
        ]
        for cp in pubs:
            cp.wait()
        plsc.subcore_barrier()

        nbase = s * nslice
        copies = []
        for t in range(nt):
            for k in range(3):
                copies.append(pltpu.async_copy(
                    sh_ref.at[t, pl.ds(k * n_pad + nbase, nslice)],
                    gath_v.at[t * 3 + k], sem))
        for cp in copies:
            cp.wait()

        def comb(j, _):
            sl = pl.ds(j * LANES, LANES)
            mm = gath_v[0, sl]
            for t in range(1, nt):
                mm = jnp.maximum(mm, gath_v[t * 3, sl])
            dacc = jnp.zeros((LANES,), jnp.float32)
            nacc = jnp.zeros((LANES,), jnp.float32)
            for t in range(nt):
                sc = jnp.exp(gath_v[t * 3, sl] - mm)
                dacc = dacc + gath_v[t * 3 + 1, sl] * sc
                nacc = nacc + gath_v[t * 3 + 2, sl] * sc
            macc_v[sl] = mm
            dacc_v[sl] = dacc
            nacc_v[sl] = nacc
            return 0
        lax.fori_loop(0, nvec, comb, 0)

        outs = [
            pltpu.async_copy(macc_v, m_part_hbm.at[c, pl.ds(nbase, nslice)],
                             sem),
            pltpu.async_copy(dacc_v,
                             den_part_hbm.at[c, pl.ds(nbase, nslice)], sem),
            pltpu.async_copy(nacc_v,
                             num_part_hbm.at[c, pl.ds(nbase, nslice)], sem),
        ]
        for cp in outs:
            cp.wait()

    return sc_kernel


def kernel(F, edge_index, W, att_src, att_dst, bias):
    n, d = F.shape
    e = edge_index.shape[1]
    nw, nt = 32, 16
    n_pad = ((n + 16 * nt - 1) // (16 * nt)) * (16 * nt)
    e_per = e // nw

    x2 = pl.pallas_call(
        _matvec_body,
        out_shape=jax.ShapeDtypeStruct((n_pad, 1), jnp.float32),
    )(F, W, att_src.reshape(1, 1), att_dst.reshape(1, 1))
    x_flat = x2.reshape(n_pad)

    sc_kernel = _make_sc_kernel(n_pad, e_per, nt, e, n)
    m_part, den_part, num_part = sc_kernel(x_flat, edge_index.reshape(2 * e))

    out2 = pl.pallas_call(
        _final_body,
        out_shape=jax.ShapeDtypeStruct((n_pad // 128, 128), jnp.float32),
    )(m_part.reshape(2, n_pad // 128, 128),
      den_part.reshape(2, n_pad // 128, 128),
      num_part.reshape(2, n_pad // 128, 128),
      bias.reshape(1, 1))
    return out2.reshape(n_pad)[:n]


# speedup vs baseline: 193.9745x; 1.0328x over previous
import functools

import jax
import jax.numpy as jnp
from jax import lax
from jax.experimental import pallas as pl
from jax.experimental.pallas import tpu as pltpu
from jax.experimental.pallas import tpu_sc as plsc

NEG_INIT = -3.0e38
LANES = 16
UNROLL = 5


def _take(v, idx):
    return lax.gather(
        v, idx[:, None],
        dimension_numbers=lax.GatherDimensionNumbers(
            offset_dims=(), collapsed_slice_dims=(0,),
            start_index_map=(0,)),
        slice_sizes=(1,),
        mode=lax.GatherScatterMode.PROMISE_IN_BOUNDS)


def _splat(v, j):
    return _take(v, jnp.zeros((LANES,), jnp.int32) + j)


def _matvec_body(f_ref, w_ref, as_ref, ad_ref, o_ref):
    n = f_ref.shape[0]
    o_ref[pl.ds(0, n), :] = jnp.dot(f_ref[:, :], w_ref[:, :],
                                    preferred_element_type=jnp.float32)
    o_ref[pl.ds(n, 8), :] = jnp.full((8, 1), 1.0) * as_ref[0, 0]
    o_ref[pl.ds(n + 8, 8), :] = jnp.full((8, 1), 1.0) * ad_ref[0, 0]


def _final_body(m_ref, den_ref, num_ref, b_ref, o_ref):
    m0 = m_ref[0]
    m1 = m_ref[1]
    mm = jnp.maximum(m0, m1)
    s0 = jnp.exp(m0 - mm)
    s1 = jnp.exp(m1 - mm)
    den = den_ref[0] * s0 + den_ref[1] * s1
    num = num_ref[0] * s0 + num_ref[1] * s1
    o_ref[:, :] = num / (den + 1e-16) + b_ref[0, 0]


def _islast(keys, iota):
    kn = _take(keys, jnp.minimum(iota + 1, LANES - 1))
    return (iota == LANES - 1) | (keys != kn)


def _seg_max_scan(keys, vals, iota):
    for sh in (1, 2, 4, 8):
        idx = jnp.maximum(iota - sh, 0)
        same = (_take(keys, idx) == keys) & (iota >= sh)
        vals = jnp.where(same, jnp.maximum(vals, _take(vals, idx)), vals)
    return vals


def _seg_sum_scan2(keys, v1, v2, iota):
    for sh in (1, 2, 4, 8):
        idx = jnp.maximum(iota - sh, 0)
        same = (_take(keys, idx) == keys) & (iota >= sh)
        v1 = jnp.where(same, v1 + _take(v1, idx), v1)
        v2 = jnp.where(same, v2 + _take(v2, idx), v2)
    return v1, v2


def _make_sc_kernel(n_pad, e_per, nt, n_edges, n_real):
    nslice = n_pad // nt
    nvec = nslice // LANES
    mesh = plsc.VectorSubcoreMesh(core_axis_name="c", subcore_axis_name="s")
    part = jax.ShapeDtypeStruct((2, n_pad), jnp.float32)

    @functools.partial(
        pl.kernel, mesh=mesh,
        compiler_params=pltpu.CompilerParams(needs_layout_passes=False),
        out_type=[part, part, part],
        scratch_types=[
            pltpu.VMEM((n_pad,), jnp.float32),
            pltpu.VMEM((e_per,), jnp.int32),
            pltpu.VMEM((e_per,), jnp.int32),
            pltpu.VMEM((n_pad,), jnp.float32),
            pltpu.VMEM((n_pad,), jnp.float32),
            pltpu.VMEM((n_pad,), jnp.float32),
            pltpu.VMEM((nt * 3, nslice), jnp.float32),
            pltpu.VMEM((nslice,), jnp.float32),
            pltpu.VMEM((nslice,), jnp.float32),
            pltpu.VMEM((nslice,), jnp.float32),
            pltpu.VMEM_SHARED((nt, 3 * n_pad), jnp.float32),
            pltpu.VMEM_SHARED((n_pad,), jnp.float32),
            pltpu.SemaphoreType.DMA,
        ],
    )
    def sc_kernel(x_hbm, edge_hbm,
                  m_part_hbm, den_part_hbm, num_part_hbm,
                  x_v, src_v, dst_v, amax_v, den_v, num_v,
                  gath_v, macc_v, dacc_v, nacc_v, sh_ref, xsh_ref, sem):
        c = lax.axis_index("c")
        s = lax.axis_index("s")
        wid = c * nt + s
        ebase = wid * e_per
        @pl.when(s == 0)
        def _stage_x():
            pltpu.sync_copy(x_hbm, xsh_ref)
        ecopies = [
            pltpu.async_copy(edge_hbm.at[pl.ds(ebase, e_per)], src_v, sem),
            pltpu.async_copy(edge_hbm.at[pl.ds(n_edges + ebase, e_per)],
                             dst_v, sem),
        ]
        plsc.subcore_barrier()
        pltpu.sync_copy(xsh_ref, x_v)
        for cp in ecopies:
            cp.wait()
        zero16 = jnp.zeros((LANES,), jnp.int32)
        a_s = plsc.load_gather(x_v, [zero16 + n_real])
        a_d = plsc.load_gather(x_v, [zero16 + (n_real + 8)])
        iota = lax.iota(jnp.int32, LANES)

        def fill(refs, value):
            fu = 8
            def body(i, _):
                v = jnp.full((LANES,), value, jnp.float32)
                for ref in refs:
                    for u in range(fu):
                        ref[pl.ds((i * fu + u) * LANES, LANES)] = v
                return 0
            lax.fori_loop(0, n_pad // (LANES * fu), body, 0)

        fill([amax_v], NEG_INIT)

        def pass1(i, _):
            base = i * (LANES * UNROLL)
            offs = [base + u * LANES for u in range(UNROLL)]
            sid = [src_v[pl.ds(o, LANES)] for o in offs]
            did = [dst_v[pl.ds(o, LANES)] for o in offs]
            kss = []
            alphas = []
            masks = []
            for u in range(UNROLL):
                ku, ss = plsc.sort_key_val(plsc.bitcast(did[u], jnp.uint32),
                                           sid[u])
                ks = plsc.bitcast(ku, jnp.int32)
                xs = plsc.load_gather(x_v, [ss])
                xd = plsc.load_gather(x_v, [ks])
                l = a_s * xs + a_d * xd
                alpha = jnp.maximum(l, 0.2 * l)
                kss.append((ks, ss))
                alphas.append(alpha)
                masks.append(_islast(ks, iota))
            for o, (ks, ss) in zip(offs, kss):
                dst_v[pl.ds(o, LANES)] = ks
                src_v[pl.ds(o, LANES)] = ss
            nodup = jnp.all(functools.reduce(lambda a, b: a & b, masks))
            vss = lax.cond(
                nodup,
                lambda: tuple(alphas),
                lambda: tuple(_seg_max_scan(kss[u][0], alphas[u], iota)
                              for u in range(UNROLL)))
            for (ks, ss), vs, islast in zip(kss, vss, masks):
                cur = plsc.load_gather(amax_v, [ks])
                plsc.store_scatter(amax_v, [ks], jnp.maximum(cur, vs),
                                   mask=islast)
            return 0
        lax.fori_loop(0, e_per // (LANES * UNROLL), pass1, 0)

        fill([den_v, num_v], 0.0)

        def pass2(i, _):
            base = i * (LANES * UNROLL)
            offs = [base + u * LANES for u in range(UNROLL)]
            kslist = []
            evals = []
            masks = []
            for o in offs:
                ks = dst_v[pl.ds(o, LANES)]
                ss = src_v[pl.ds(o, LANES)]
                xs = plsc.load_gather(x_v, [ss])
                xd = plsc.load_gather(x_v, [ks])
                l = a_s * xs + a_d * xd
                alpha = jnp.maximum(l, 0.2 * l)
                am = plsc.load_gather(amax_v, [ks])
                e = jnp.exp(alpha - am)
                kslist.append(ks)
                evals.append((e, e * xs))
                masks.append(_islast(ks, iota))
            nodup = jnp.all(functools.reduce(lambda a, b: a & b, masks))

            def scanned():
                out = []
                for u in range(UNROLL):
                    es, xxs = _seg_sum_scan2(kslist[u], evals[u][0],
                                             evals[u][1], iota)
                    out.extend((es, xxs))
                return tuple(out)

            flat = lax.cond(
                nodup,
                lambda: tuple(v for ev in evals for v in ev),
                scanned)
            for u in range(UNROLL):
                plsc.addupdate_scatter(den_v, [kslist[u]], flat[2 * u],
                                       mask=masks[u])
                plsc.addupdate_scatter(num_v, [kslist[u]],
                                       flat[2 * u + 1], mask=masks[u])
            return 0
        lax.fori_loop(0, e_per // (LANES * UNROLL), pass2, 0)

        pubs = [
            pltpu.async_copy(amax_v, sh_ref.at[s, pl.ds(0, n_pad)], sem),
            pltpu.async_copy(den_v, sh_ref.at[s, pl.ds(n_pad, n_pad)], sem),
            pltpu.async_copy(num_v, sh_ref.at[s, pl.ds(2 * n_pad, n_pad)],
                             sem),
        ]
        for cp in pubs:
            cp.wait()
        plsc.subcore_barrier()

        nbase = s * nslice
        copies = []
        for t in range(nt):
            for k in range(3):
                copies.append(pltpu.async_copy(
                    sh_ref.at[t, pl.ds(k * n_pad + nbase, nslice)],
                    gath_v.at[t * 3 + k], sem))
        for cp in copies:
            cp.wait()

        def comb(j, _):
            sl = pl.ds(j * LANES, LANES)
            mm = gath_v[0, sl]
            for t in range(1, nt):
                mm = jnp.maximum(mm, gath_v[t * 3, sl])
            dacc = jnp.zeros((LANES,), jnp.float32)
            nacc = jnp.zeros((LANES,), jnp.float32)
            for t in range(nt):
                sc = jnp.exp(gath_v[t * 3, sl] - mm)
                dacc = dacc + gath_v[t * 3 + 1, sl] * sc
                nacc = nacc + gath_v[t * 3 + 2, sl] * sc
            macc_v[sl] = mm
            dacc_v[sl] = dacc
            nacc_v[sl] = nacc
            return 0
        lax.fori_loop(0, nvec, comb, 0)

        outs = [
            pltpu.async_copy(macc_v, m_part_hbm.at[c, pl.ds(nbase, nslice)],
                             sem),
            pltpu.async_copy(dacc_v,
                             den_part_hbm.at[c, pl.ds(nbase, nslice)], sem),
            pltpu.async_copy(nacc_v,
                             num_part_hbm.at[c, pl.ds(nbase, nslice)], sem),
        ]
        for cp in outs:
            cp.wait()

    return sc_kernel


def kernel(F, edge_index, W, att_src, att_dst, bias):
    n, d = F.shape
    e = edge_index.shape[1]
    nw, nt = 32, 16
    n_pad = ((n + 16 * nt - 1) // (16 * nt)) * (16 * nt)
    e_per = e // nw

    x2 = pl.pallas_call(
        _matvec_body,
        out_shape=jax.ShapeDtypeStruct((n_pad, 1), jnp.float32),
    )(F, W, att_src.reshape(1, 1), att_dst.reshape(1, 1))
    x_flat = x2.reshape(n_pad)

    sc_kernel = _make_sc_kernel(n_pad, e_per, nt, e, n)
    m_part, den_part, num_part = sc_kernel(x_flat, edge_index.reshape(2 * e))

    out2 = pl.pallas_call(
        _final_body,
        out_shape=jax.ShapeDtypeStruct((n_pad // 128, 128), jnp.float32),
    )(m_part.reshape(2, n_pad // 128, 128),
      den_part.reshape(2, n_pad // 128, 128),
      num_part.reshape(2, n_pad // 128, 128),
      bias.reshape(1, 1))
    return out2.reshape(n_pad)[:n]
